# baseline (speedup 1.0000x reference)
import functools

import jax
import jax.numpy as jnp
from jax import lax
from jax.experimental import pallas as pl
from jax.experimental.pallas import tpu as pltpu
from jax.experimental.pallas import tpu_sc as plsc

_VOCAB = 100000
_DIM = 16
_B = 16384
_F = 26
_NF = 25
_NMETA = 4
_GCOLS = _NF * _DIM
_HIDDEN = 64

_NW = 32
_RB = _B // _NW
_CHUNK = 128
_NCH = _RB // _CHUNK
_TB = 2048


def _sc_gather(inputs, t128):
    mesh = plsc.VectorSubcoreMesh(core_axis_name="c", subcore_axis_name="s")
    nc = mesh.num_cores
    nlane = _RB // 16

    @functools.partial(
        pl.kernel,
        out_type=jax.ShapeDtypeStruct((_B, _GCOLS), jnp.float32),
        mesh=mesh,
        scratch_types=[
            pltpu.VMEM((_RB, _F), jnp.int32),
            pltpu.VMEM((_RB,), jnp.int32),
            pltpu.VMEM((_RB,), jnp.int32),
            pltpu.VMEM((_RB, 128), jnp.float32),
            pltpu.VMEM((_RB, _DIM), jnp.float32),
            pltpu.SemaphoreType.DMA,
        ],
        compiler_params=pltpu.CompilerParams(
            use_tc_tiling_on_sc=False, needs_layout_passes=False
        ),
    )
    def k(in_hbm, tab_hbm, out_hbm, blk_v, sup_v, off_v, chunk_v, rows_v, sem):
        wid = lax.axis_index("s") * nc + lax.axis_index("c")
        base = wid * _RB
        pltpu.sync_copy(in_hbm.at[pl.ds(base, _RB), :], blk_v)

        @pl.loop(0, _NF)
        def _field(f):
            fld = f + 1
            col = jnp.full((16,), fld, dtype=jnp.int32)
            for j in range(nlane):
                rows = lax.iota(jnp.int32, 16) + (16 * j)
                vals = plsc.load_gather(blk_v, [rows, col])
                sup_v[pl.ds(j * 16, 16)] = lax.shift_right_logical(vals, 3)
                off_v[pl.ds(j * 16, 16)] = (vals & 7) * _DIM
            copies = [
                pltpu.async_copy(
                    tab_hbm.at[fld].at[sup_v.at[pl.ds(c * _CHUNK, _CHUNK)]],
                    chunk_v.at[pl.ds(c * _CHUNK, _CHUNK)],
                    sem,
                )
                for c in range(_NCH)
            ]
            for cp in copies:
                cp.wait()

            @pl.loop(0, nlane)
            def _extract(j):
                offs = off_v[pl.ds(j * 16, 16)]
                for jj in range(16):
                    r = j * 16 + jj
                    rows_v[r, :] = chunk_v[r, pl.ds(offs[jj], _DIM)]

            pltpu.sync_copy(
                rows_v, out_hbm.at[pl.ds(base, _RB), pl.ds(f * _DIM, _DIM)]
            )

    return k(inputs, t128)


def _mlp_body(g_ref, w1_ref, b1_ref, w2_ref, b2_ref, out_ref):
    z = (
        jnp.dot(g_ref[...], w1_ref[...], preferred_element_type=jnp.float32)
        + b1_ref[...]
    )
    h1 = jnp.maximum(z, 0.0)
    p = jnp.dot(h1, w2_ref[...], preferred_element_type=jnp.float32) + b2_ref[...]
    out_ref[...] = 1.0 / (1.0 + jnp.exp(-p))


def _tc_mlp(g, w1_eff, b_eff, w2t, b2):
    return pl.pallas_call(
        _mlp_body,
        grid=(_B // _TB,),
        in_specs=[
            pl.BlockSpec((_TB, _GCOLS), lambda i: (i, 0)),
            pl.BlockSpec((_GCOLS, _HIDDEN), lambda i: (0, 0)),
            pl.BlockSpec((1, _HIDDEN), lambda i: (0, 0)),
            pl.BlockSpec((_HIDDEN, 1), lambda i: (0, 0)),
            pl.BlockSpec((1, 1), lambda i: (0, 0)),
        ],
        out_specs=pl.BlockSpec((_TB, 1), lambda i: (i, 0)),
        out_shape=jax.ShapeDtypeStruct((_B, 1), jnp.float32),
    )(g, w1_eff, b_eff, w2t, b2)


def kernel(inputs, tables, W_meta, b_meta, W1, b1, W2, b2):
    t128 = tables.reshape(_F, _VOCAB // 8, 128)

    w1a = W1[:, :_DIM]
    w1_eff = W1[:, _DIM:].T
    mpool = jnp.repeat(
        jnp.eye(_NMETA, dtype=jnp.float32), _DIM, axis=0
    ) / _DIM
    fold = mpool @ W_meta.T @ w1a.T
    w1_eff = w1_eff.at[: _NMETA * _DIM].add(fold)
    b_eff = (b1 + b_meta @ w1a.T)[None, :]
    w2t = W2.T
    b2r = b2[None, :]

    g = _sc_gather(inputs, t128)
    return _tc_mlp(g, w1_eff, b_eff, w2t, b2r)


# speedup vs baseline: 1.0987x; 1.0987x over previous
import functools

import jax
import jax.numpy as jnp
from jax import lax
from jax.experimental import pallas as pl
from jax.experimental.pallas import tpu as pltpu
from jax.experimental.pallas import tpu_sc as plsc

_VOCAB = 100000
_DIM = 16
_B = 16384
_F = 26
_NF = 25
_NMETA = 4
_GCOLS = _NF * _DIM
_HIDDEN = 64

_NW = 32
_RB = _B // _NW
_CHUNK = 128
_NCH = _RB // _CHUNK
_TB = 2048


def _sc_gather(inputs, tables):
    mesh = plsc.VectorSubcoreMesh(core_axis_name="c", subcore_axis_name="s")
    nc = mesh.num_cores
    nlane = _RB // 16

    @functools.partial(
        pl.kernel,
        out_type=jax.ShapeDtypeStruct((_B, _GCOLS), jnp.float32),
        mesh=mesh,
        scratch_types=[
            pltpu.VMEM((_RB, _F), jnp.int32),
            pltpu.VMEM((_RB,), jnp.int32),
            pltpu.VMEM((_RB, _DIM), jnp.float32),
            pltpu.SemaphoreType.DMA,
        ],
        compiler_params=pltpu.CompilerParams(
            use_tc_tiling_on_sc=False, needs_layout_passes=False
        ),
    )
    def k(in_hbm, tab_hbm, out_hbm, blk_v, idx_v, rows_v, sem):
        wid = lax.axis_index("s") * nc + lax.axis_index("c")
        base = wid * _RB
        pltpu.sync_copy(in_hbm.at[pl.ds(base, _RB), :], blk_v)

        @pl.loop(0, _NF)
        def _field(f):
            fld = f + 1
            col = jnp.full((16,), fld, dtype=jnp.int32)
            for j in range(nlane):
                rows = lax.iota(jnp.int32, 16) + (16 * j)
                vals = plsc.load_gather(blk_v, [rows, col])
                idx_v[pl.ds(j * 16, 16)] = vals
            copies = [
                pltpu.async_copy(
                    tab_hbm.at[fld].at[idx_v.at[pl.ds(c * _CHUNK, _CHUNK)]],
                    rows_v.at[pl.ds(c * _CHUNK, _CHUNK)],
                    sem,
                )
                for c in range(_NCH)
            ]
            for cp in copies:
                cp.wait()
            pltpu.sync_copy(
                rows_v, out_hbm.at[pl.ds(base, _RB), pl.ds(f * _DIM, _DIM)]
            )

    return k(inputs, tables)


def _mlp_body(g_ref, w1_ref, b1_ref, w2_ref, b2_ref, out_ref):
    z = (
        jnp.dot(g_ref[...], w1_ref[...], preferred_element_type=jnp.float32)
        + b1_ref[...]
    )
    h1 = jnp.maximum(z, 0.0)
    p = jnp.dot(h1, w2_ref[...], preferred_element_type=jnp.float32) + b2_ref[...]
    out_ref[...] = 1.0 / (1.0 + jnp.exp(-p))


def _tc_mlp(g, w1_eff, b_eff, w2t, b2):
    return pl.pallas_call(
        _mlp_body,
        grid=(_B // _TB,),
        in_specs=[
            pl.BlockSpec((_TB, _GCOLS), lambda i: (i, 0)),
            pl.BlockSpec((_GCOLS, _HIDDEN), lambda i: (0, 0)),
            pl.BlockSpec((1, _HIDDEN), lambda i: (0, 0)),
            pl.BlockSpec((_HIDDEN, 1), lambda i: (0, 0)),
            pl.BlockSpec((1, 1), lambda i: (0, 0)),
        ],
        out_specs=pl.BlockSpec((_TB, 1), lambda i: (i, 0)),
        out_shape=jax.ShapeDtypeStruct((_B, 1), jnp.float32),
    )(g, w1_eff, b_eff, w2t, b2)


def kernel(inputs, tables, W_meta, b_meta, W1, b1, W2, b2):
    w1a = W1[:, :_DIM]
    w1_eff = W1[:, _DIM:].T
    mpool = jnp.repeat(
        jnp.eye(_NMETA, dtype=jnp.float32), _DIM, axis=0
    ) / _DIM
    fold = mpool @ W_meta.T @ w1a.T
    w1_eff = w1_eff.at[: _NMETA * _DIM].add(fold)
    b_eff = (b1 + b_meta @ w1a.T)[None, :]
    w2t = W2.T
    b2r = b2[None, :]

    g = _sc_gather(inputs, tables)
    return _tc_mlp(g, w1_eff, b_eff, w2t, b2r)
